# baseline (speedup 1.0000x reference)
import jax
import jax.numpy as jnp
from jax import lax
from jax.experimental import pallas as pl
from jax.experimental.pallas import tpu as pltpu
from jax.experimental.pallas import tpu_sc as plsc
import functools

_NC = 2
_NS = 16
_NW = _NC * _NS
_CHUNK = 32


def _sc_body(nchunks, chunk_elems, seq_elems, x_hbm, pos_hbm, out_hbm,
             bufx, bufp, sx0, sx1, sp, so0, so1):
    cid = lax.axis_index("c")
    sid = lax.axis_index("s")
    wid = sid * _NC + cid
    elems_per_w = nchunks * chunk_elems
    base = wid * elems_per_w
    pbase = lax.rem(base, seq_elems)
    nvec = chunk_elems // 16
    sx = (sx0, sx1)
    so = (so0, so1)
    x_cp = [None] * nchunks
    p_cp = [None] * nchunks
    out_cp = [None] * nchunks
    x_cp[0] = pltpu.async_copy(
        x_hbm.at[pl.ds(base, chunk_elems)], bufx.at[0], sx[0])
    p_cp[0] = pltpu.async_copy(
        pos_hbm.at[pl.ds(pbase, chunk_elems)], bufp, sp)
    for c in range(nchunks):
        s = c & 1
        if c + 1 < nchunks:
            if c >= 1:
                out_cp[c - 1].wait()
            x_cp[c + 1] = pltpu.async_copy(
                x_hbm.at[pl.ds(base + (c + 1) * chunk_elems, chunk_elems)],
                bufx.at[1 - s], sx[1 - s])
        x_cp[c].wait()
        p_cp[c].wait()

        @pl.loop(0, nvec, unroll=8)
        def _(i):
            off = pl.multiple_of(i * 16, 16)
            plsc.addupdate(bufx.at[s, pl.ds(off, 16)], bufp[pl.ds(off, 16)])

        if c + 1 < nchunks:
            p_cp[c + 1] = pltpu.async_copy(
                pos_hbm.at[pl.ds(pbase + (c + 1) * chunk_elems, chunk_elems)],
                bufp, sp)
        out_cp[c] = pltpu.async_copy(
            bufx.at[s], out_hbm.at[pl.ds(base + c * chunk_elems, chunk_elems)],
            so[s])
    out_cp[nchunks - 1].wait()
    if nchunks >= 2:
        out_cp[nchunks - 2].wait()


def _sc_add(x1, pos1, hid):
    total = x1.shape[0]
    chunk_elems = _CHUNK * hid
    elems_per_w = total // _NW
    nchunks = elems_per_w // chunk_elems
    mesh = plsc.VectorSubcoreMesh(core_axis_name="c", subcore_axis_name="s")
    return pl.kernel(
        functools.partial(_sc_body, nchunks, chunk_elems, pos1.shape[0]),
        out_type=jax.ShapeDtypeStruct((total,), x1.dtype),
        mesh=mesh,
        scratch_types=[
            pltpu.VMEM((2, chunk_elems), jnp.float32),
            pltpu.VMEM((chunk_elems,), jnp.float32),
            pltpu.SemaphoreType.DMA,
            pltpu.SemaphoreType.DMA,
            pltpu.SemaphoreType.DMA,
            pltpu.SemaphoreType.DMA,
            pltpu.SemaphoreType.DMA,
        ],
    )(x1, pos1)


def kernel(x, pos_embedding):
    batch, seq, hid = x.shape
    x1 = x.reshape(batch * seq * hid)
    pos1 = pos_embedding.reshape(seq * hid)
    out1 = _sc_add(x1, pos1, hid)
    return out1.reshape(batch, seq, hid)


_BLK_S = 2048


def _add_body(x_ref, pos_ref, o_ref):
    o_ref[...] = x_ref[...] + pos_ref[...][None, :, :]


def _tc_kernel(x, pos_embedding):
    batch, seq, hid = x.shape
    grid = (seq // _BLK_S, batch)
    return pl.pallas_call(
        _add_body,
        grid=grid,
        in_specs=[
            pl.BlockSpec((1, _BLK_S, hid), lambda s, b: (b, s, 0)),
            pl.BlockSpec((_BLK_S, hid), lambda s, b: (s, 0)),
        ],
        out_specs=pl.BlockSpec((1, _BLK_S, hid), lambda s, b: (b, s, 0)),
        out_shape=jax.ShapeDtypeStruct(x.shape, x.dtype),
        compiler_params=pltpu.CompilerParams(
            dimension_semantics=("arbitrary", "arbitrary"),
        ),
    )(x, pos_embedding)


# speedup vs baseline: 1.9032x; 1.9032x over previous
import jax
import jax.numpy as jnp
from jax import lax
from jax.experimental import pallas as pl
from jax.experimental.pallas import tpu as pltpu
from jax.experimental.pallas import tpu_sc as plsc
import functools

_NC = 2
_NS = 16
_NW = _NC * _NS
_CHUNK = 32


def _sc_body(nchunks, hid, x_hbm, pos_hbm, out_hbm,
             bufx, bufp, sx0, sx1, sp, so0, so1):
    cid = lax.axis_index("c")
    sid = lax.axis_index("s")
    wid = sid * _NC + cid
    rows_per_w = nchunks * _CHUNK
    base = wid * rows_per_w
    pbase = lax.rem(base, pos_hbm.shape[0])
    per_row = hid // 16
    nvec = _CHUNK * per_row
    sx = (sx0, sx1)
    so = (so0, so1)
    x_cp = [None] * nchunks
    p_cp = [None] * nchunks
    out_cp = [None] * nchunks
    x_cp[0] = pltpu.async_copy(x_hbm.at[pl.ds(base, _CHUNK)], bufx.at[0], sx[0])
    p_cp[0] = pltpu.async_copy(pos_hbm.at[pl.ds(pbase, _CHUNK)], bufp, sp)
    for c in range(nchunks):
        s = c & 1
        if c + 1 < nchunks:
            if c >= 1:
                out_cp[c - 1].wait()
            x_cp[c + 1] = pltpu.async_copy(
                x_hbm.at[pl.ds(base + (c + 1) * _CHUNK, _CHUNK)],
                bufx.at[1 - s], sx[1 - s])
        x_cp[c].wait()
        p_cp[c].wait()

        @pl.loop(0, nvec, unroll=8)
        def _(i):
            r = i // per_row
            off = pl.multiple_of((i % per_row) * 16, 16)
            plsc.addupdate(bufx.at[s, r, pl.ds(off, 16)], bufp[r, pl.ds(off, 16)])

        if c + 1 < nchunks:
            p_cp[c + 1] = pltpu.async_copy(
                pos_hbm.at[pl.ds(pbase + (c + 1) * _CHUNK, _CHUNK)], bufp, sp)
        out_cp[c] = pltpu.async_copy(
            bufx.at[s], out_hbm.at[pl.ds(base + c * _CHUNK, _CHUNK)], so[s])
    out_cp[nchunks - 1].wait()
    if nchunks >= 2:
        out_cp[nchunks - 2].wait()


def _sc_add(x2, pos_embedding):
    rows, hid = x2.shape
    rows_per_w = rows // _NW
    nchunks = rows_per_w // _CHUNK
    mesh = plsc.VectorSubcoreMesh(core_axis_name="c", subcore_axis_name="s")
    return pl.kernel(
        functools.partial(_sc_body, nchunks, hid),
        out_type=jax.ShapeDtypeStruct((rows, hid), x2.dtype),
        mesh=mesh,
        scratch_types=[
            pltpu.VMEM((2, _CHUNK, hid), jnp.float32),
            pltpu.VMEM((_CHUNK, hid), jnp.float32),
            pltpu.SemaphoreType.DMA,
            pltpu.SemaphoreType.DMA,
            pltpu.SemaphoreType.DMA,
            pltpu.SemaphoreType.DMA,
            pltpu.SemaphoreType.DMA,
        ],
    )(x2, pos_embedding)


def kernel(x, pos_embedding):
    batch, seq, hid = x.shape
    x2 = x.reshape(batch * seq, hid)
    out2 = _sc_add(x2, pos_embedding)
    return out2.reshape(batch, seq, hid)


_BLK_S = 2048


def _add_body(x_ref, pos_ref, o_ref):
    o_ref[...] = x_ref[...] + pos_ref[...][None, :, :]


def _tc_kernel(x, pos_embedding):
    batch, seq, hid = x.shape
    grid = (seq // _BLK_S, batch)
    return pl.pallas_call(
        _add_body,
        grid=grid,
        in_specs=[
            pl.BlockSpec((1, _BLK_S, hid), lambda s, b: (b, s, 0)),
            pl.BlockSpec((_BLK_S, hid), lambda s, b: (s, 0)),
        ],
        out_specs=pl.BlockSpec((1, _BLK_S, hid), lambda s, b: (b, s, 0)),
        out_shape=jax.ShapeDtypeStruct(x.shape, x.dtype),
        compiler_params=pltpu.CompilerParams(
            dimension_semantics=("arbitrary", "arbitrary"),
        ),
    )(x, pos_embedding)
